# baseline (speedup 1.0000x reference)
import functools

import jax
import jax.numpy as jnp
from jax import lax
from jax.experimental import pallas as pl
from jax.experimental.pallas import tpu as pltpu
from jax.experimental.pallas import tpu_sc as plsc

N_NODES = 10000
IN_F = 128
OUT_F = 128
N_EDGES = 320000

NC = 2
NS = 16
L = 16
NW = NC * NS
FPT = IN_F // NW
PPT = FPT // 2
HALF = IN_F // 2
CHUNK = 3200
NCHUNK = N_EDGES // CHUNK
GROUPS = CHUNK // L
TBL = PPT * N_NODES
ACC = FPT * N_NODES


def _pack_body(i_ref, x_ref, o_ref):
    xt = lax.dot_general(
        i_ref[...],
        x_ref[...],
        dimension_numbers=(((1,), (1,)), ((), ())),
        preferred_element_type=jnp.float32,
        precision=lax.Precision.HIGHEST,
    )
    lo = lax.bitcast_convert_type(
        xt[:HALF].astype(jnp.bfloat16), jnp.uint16
    ).astype(jnp.uint32)
    hi = lax.bitcast_convert_type(
        xt[HALF:].astype(jnp.bfloat16), jnp.uint16
    ).astype(jnp.uint32)
    o_ref[...] = lax.bitcast_convert_type(lo | (hi << 16), jnp.int32)


def _x_packed(x):
    n = x.shape[0]
    eye = jnp.eye(IN_F, dtype=jnp.float32)
    return pl.pallas_call(
        _pack_body,
        out_shape=jax.ShapeDtypeStruct((HALF, n), jnp.int32),
    )(eye, x)


def _out_body(aggt_ref, w_ref, b_ref, o_ref):
    o_ref[...] = lax.dot_general(
        aggt_ref[...],
        w_ref[...],
        dimension_numbers=(((0,), (0,)), ((), ())),
        preferred_element_type=jnp.float32,
        precision=lax.Precision.HIGHEST,
    ) + b_ref[...][None, :]


def _out_matmul(aggt, W, b):
    return pl.pallas_call(
        _out_body,
        out_shape=jax.ShapeDtypeStruct((N_NODES, OUT_F), jnp.float32),
    )(aggt, W, b)


_mesh = plsc.VectorSubcoreMesh(
    core_axis_name="c", subcore_axis_name="s", num_cores=NC, num_subcores=NS
)


@functools.partial(
    pl.kernel,
    out_type=jax.ShapeDtypeStruct((IN_F * N_NODES,), jnp.float32),
    mesh=_mesh,
    compiler_params=pltpu.CompilerParams(needs_layout_passes=False),
    scratch_types=[
        pltpu.VMEM((TBL,), jnp.int32),
        pltpu.VMEM((ACC,), jnp.float32),
        pltpu.VMEM((CHUNK,), jnp.int32),
        pltpu.VMEM((CHUNK,), jnp.float32),
        pltpu.VMEM((CHUNK,), jnp.int32),
        pltpu.VMEM((CHUNK,), jnp.float32),
        pltpu.SemaphoreType.DMA,
        pltpu.SemaphoreType.DMA,
        pltpu.SemaphoreType.DMA,
        pltpu.SemaphoreType.DMA,
    ],
)
def _sc_agg(sup_hbm, pidx_hbm, ew_hbm, out_hbm,
            table_v, acc_v,
            pidx0, ew0, pidx1, ew1,
            sem_p0, sem_w0, sem_p1, sem_w1):
    cid = lax.axis_index("c")
    sid = lax.axis_index("s")
    wid = sid * NC + cid

    pltpu.sync_copy(sup_hbm.at[pl.ds(wid * TBL, TBL)], table_v)

    zvec = jnp.zeros((L,), jnp.float32)

    @pl.loop(0, ACC // L)
    def _init(i):
        acc_v[pl.ds(i * L, L)] = zvec

    slots = (
        (pidx0, ew0, sem_p0, sem_w0),
        (pidx1, ew1, sem_p1, sem_w1),
    )

    def start(c, slot):
        p_b, w_b, p_s, w_s = slot
        off = c * CHUNK
        pltpu.make_async_copy(pidx_hbm.at[pl.ds(off, CHUNK)], p_b, p_s).start()
        pltpu.make_async_copy(ew_hbm.at[pl.ds(off, CHUNK)], w_b, w_s).start()

    def wait(slot):
        p_b, w_b, p_s, w_s = slot
        pltpu.make_async_copy(pidx_hbm.at[pl.ds(0, CHUNK)], p_b, p_s).wait()
        pltpu.make_async_copy(ew_hbm.at[pl.ds(0, CHUNK)], w_b, w_s).wait()

    def process(slot):
        p_b, w_b = slot[:2]

        @plsc.parallel_loop(0, GROUPS, unroll=4)
        def _grp(g):
            o = g * L
            p = p_b[pl.ds(o, L)]
            w = w_b[pl.ds(o, L)]
            s = p & 0xFFFF
            d = lax.shift_right_logical(p, 16)
            for fp in range(PPT):
                si = s if fp == 0 else s + fp * N_NODES
                vp = plsc.load_gather(table_v, [si])
                vlo = plsc.bitcast(lax.shift_left(vp, 16), jnp.float32)
                vhi = plsc.bitcast(vp & jnp.int32(-65536), jnp.float32)
                dlo = d if fp == 0 else d + fp * N_NODES
                plsc.addupdate_scatter(acc_v, [dlo], vlo * w)
                plsc.addupdate_scatter(
                    acc_v, [d + (2 + fp) * N_NODES], vhi * w
                )

    start(0, slots[0])
    start(1, slots[1])

    @pl.loop(0, NCHUNK, step=2)
    def _chunk(c):
        wait(slots[0])
        process(slots[0])

        @pl.when(c + 2 < NCHUNK)
        def _():
            start(c + 2, slots[0])

        wait(slots[1])
        process(slots[1])

        @pl.when(c + 3 < NCHUNK)
        def _():
            start(c + 3, slots[1])

    pltpu.sync_copy(
        acc_v.at[pl.ds(0, 2 * N_NODES)],
        out_hbm.at[pl.ds(2 * wid * N_NODES, 2 * N_NODES)],
    )
    pltpu.sync_copy(
        acc_v.at[pl.ds(2 * N_NODES, 2 * N_NODES)],
        out_hbm.at[pl.ds((HALF + 2 * wid) * N_NODES, 2 * N_NODES)],
    )


def kernel(x, edge_index, edge_weight, W, b):
    src = edge_index[0].astype(jnp.int32)
    dst = edge_index[1].astype(jnp.int32)
    packed = src | (dst << 16)
    x_packed = _x_packed(x)
    aggt_flat = _sc_agg(
        x_packed.reshape(-1), packed, edge_weight.astype(jnp.float32)
    )
    return _out_matmul(aggt_flat.reshape(IN_F, N_NODES), W, b)


# speedup vs baseline: 1.0282x; 1.0282x over previous
import functools

import jax
import jax.numpy as jnp
from jax import lax
from jax.experimental import pallas as pl
from jax.experimental.pallas import tpu as pltpu
from jax.experimental.pallas import tpu_sc as plsc

N_NODES = 10000
IN_F = 128
OUT_F = 128
N_EDGES = 320000

NC = 2
NS = 16
L = 16
NW = NC * NS
FPT = IN_F // NW
PPT = FPT // 2
HALF = IN_F // 2
CHUNK = 3200
NCHUNK = N_EDGES // CHUNK
GROUPS = CHUNK // L
TBL = PPT * N_NODES
ACC = FPT * N_NODES


def _pack_body(i_ref, x_ref, o_ref):
    xt = lax.dot_general(
        i_ref[...],
        x_ref[...],
        dimension_numbers=(((1,), (1,)), ((), ())),
        preferred_element_type=jnp.float32,
    )
    lo = lax.bitcast_convert_type(
        xt[:HALF].astype(jnp.bfloat16), jnp.uint16
    ).astype(jnp.uint32)
    hi = lax.bitcast_convert_type(
        xt[HALF:].astype(jnp.bfloat16), jnp.uint16
    ).astype(jnp.uint32)
    o_ref[...] = lax.bitcast_convert_type(lo | (hi << 16), jnp.int32)


def _x_packed(x):
    n = x.shape[0]
    eye = jnp.eye(IN_F, dtype=jnp.float32)
    return pl.pallas_call(
        _pack_body,
        out_shape=jax.ShapeDtypeStruct((HALF, n), jnp.int32),
    )(eye, x)


def _out_body(aggt_ref, w_ref, b_ref, o_ref):
    o_ref[...] = lax.dot_general(
        aggt_ref[...],
        w_ref[...],
        dimension_numbers=(((0,), (0,)), ((), ())),
        preferred_element_type=jnp.float32,
    ) + b_ref[...][None, :]


def _out_matmul(aggt, W, b):
    return pl.pallas_call(
        _out_body,
        out_shape=jax.ShapeDtypeStruct((N_NODES, OUT_F), jnp.float32),
    )(aggt, W, b)


_mesh = plsc.VectorSubcoreMesh(
    core_axis_name="c", subcore_axis_name="s", num_cores=NC, num_subcores=NS
)


@functools.partial(
    pl.kernel,
    out_type=jax.ShapeDtypeStruct((IN_F * N_NODES,), jnp.float32),
    mesh=_mesh,
    compiler_params=pltpu.CompilerParams(needs_layout_passes=False),
    scratch_types=[
        pltpu.VMEM((TBL,), jnp.int32),
        pltpu.VMEM((ACC,), jnp.float32),
        pltpu.VMEM((CHUNK,), jnp.int32),
        pltpu.VMEM((CHUNK,), jnp.float32),
        pltpu.VMEM((CHUNK,), jnp.int32),
        pltpu.VMEM((CHUNK,), jnp.float32),
        pltpu.SemaphoreType.DMA,
        pltpu.SemaphoreType.DMA,
        pltpu.SemaphoreType.DMA,
        pltpu.SemaphoreType.DMA,
    ],
)
def _sc_agg(sup_hbm, pidx_hbm, ew_hbm, out_hbm,
            table_v, acc_v,
            pidx0, ew0, pidx1, ew1,
            sem_p0, sem_w0, sem_p1, sem_w1):
    cid = lax.axis_index("c")
    sid = lax.axis_index("s")
    wid = sid * NC + cid

    pltpu.sync_copy(sup_hbm.at[pl.ds(wid * TBL, TBL)], table_v)

    zvec = jnp.zeros((L,), jnp.float32)

    @pl.loop(0, ACC // L)
    def _init(i):
        acc_v[pl.ds(i * L, L)] = zvec

    slots = (
        (pidx0, ew0, sem_p0, sem_w0),
        (pidx1, ew1, sem_p1, sem_w1),
    )

    def start(c, slot):
        p_b, w_b, p_s, w_s = slot
        off = c * CHUNK
        pltpu.make_async_copy(pidx_hbm.at[pl.ds(off, CHUNK)], p_b, p_s).start()
        pltpu.make_async_copy(ew_hbm.at[pl.ds(off, CHUNK)], w_b, w_s).start()

    def wait(slot):
        p_b, w_b, p_s, w_s = slot
        pltpu.make_async_copy(pidx_hbm.at[pl.ds(0, CHUNK)], p_b, p_s).wait()
        pltpu.make_async_copy(ew_hbm.at[pl.ds(0, CHUNK)], w_b, w_s).wait()

    def process(slot):
        p_b, w_b = slot[:2]

        @plsc.parallel_loop(0, GROUPS, unroll=4)
        def _grp(g):
            o = g * L
            p = p_b[pl.ds(o, L)]
            w = w_b[pl.ds(o, L)]
            s = p & 0xFFFF
            d = lax.shift_right_logical(p, 16)
            for fp in range(PPT):
                si = s if fp == 0 else s + fp * N_NODES
                vp = plsc.load_gather(table_v, [si])
                vlo = plsc.bitcast(lax.shift_left(vp, 16), jnp.float32)
                vhi = plsc.bitcast(vp & jnp.int32(-65536), jnp.float32)
                dlo = d if fp == 0 else d + fp * N_NODES
                plsc.addupdate_scatter(acc_v, [dlo], vlo * w)
                plsc.addupdate_scatter(
                    acc_v, [d + (2 + fp) * N_NODES], vhi * w
                )

    start(0, slots[0])
    start(1, slots[1])

    @pl.loop(0, NCHUNK, step=2)
    def _chunk(c):
        wait(slots[0])
        process(slots[0])

        @pl.when(c + 2 < NCHUNK)
        def _():
            start(c + 2, slots[0])

        wait(slots[1])
        process(slots[1])

        @pl.when(c + 3 < NCHUNK)
        def _():
            start(c + 3, slots[1])

    pltpu.sync_copy(
        acc_v.at[pl.ds(0, 2 * N_NODES)],
        out_hbm.at[pl.ds(2 * wid * N_NODES, 2 * N_NODES)],
    )
    pltpu.sync_copy(
        acc_v.at[pl.ds(2 * N_NODES, 2 * N_NODES)],
        out_hbm.at[pl.ds((HALF + 2 * wid) * N_NODES, 2 * N_NODES)],
    )


def kernel(x, edge_index, edge_weight, W, b):
    src = edge_index[0].astype(jnp.int32)
    dst = edge_index[1].astype(jnp.int32)
    packed = src | (dst << 16)
    x_packed = _x_packed(x)
    aggt_flat = _sc_agg(
        x_packed.reshape(-1), packed, edge_weight.astype(jnp.float32)
    )
    return _out_matmul(aggt_flat.reshape(IN_F, N_NODES), W, b)


# speedup vs baseline: 1.0410x; 1.0125x over previous
import functools

import jax
import jax.numpy as jnp
from jax import lax
from jax.experimental import pallas as pl
from jax.experimental.pallas import tpu as pltpu
from jax.experimental.pallas import tpu_sc as plsc

N_NODES = 10000
IN_F = 128
OUT_F = 128
N_EDGES = 320000

NC = 2
NS = 16
L = 16
NW = NC * NS
FPT = IN_F // NW
PPT = FPT // 2
HALF = IN_F // 2
CHUNK = 3200
NCHUNK = N_EDGES // CHUNK
GROUPS = CHUNK // L
TBL = PPT * N_NODES
ACC = FPT * N_NODES


def _pack_body(i_ref, x_ref, o_ref):
    xt = lax.dot_general(
        i_ref[...],
        x_ref[...],
        dimension_numbers=(((1,), (1,)), ((), ())),
        preferred_element_type=jnp.float32,
    )
    lo = lax.bitcast_convert_type(
        xt[:HALF].astype(jnp.bfloat16), jnp.uint16
    ).astype(jnp.uint32)
    hi = lax.bitcast_convert_type(
        xt[HALF:].astype(jnp.bfloat16), jnp.uint16
    ).astype(jnp.uint32)
    o_ref[...] = lax.bitcast_convert_type(lo | (hi << 16), jnp.int32)


def _x_packed(x):
    n = x.shape[0]
    eye = jnp.eye(IN_F, dtype=jnp.float32)
    return pl.pallas_call(
        _pack_body,
        out_shape=jax.ShapeDtypeStruct((HALF, n), jnp.int32),
    )(eye, x)


def _out_body(aggt_ref, w_ref, b_ref, o_ref):
    o_ref[...] = lax.dot_general(
        aggt_ref[...],
        w_ref[...],
        dimension_numbers=(((0,), (0,)), ((), ())),
        preferred_element_type=jnp.float32,
    ) + b_ref[...][None, :]


def _out_matmul(aggt, W, b):
    return pl.pallas_call(
        _out_body,
        out_shape=jax.ShapeDtypeStruct((N_NODES, OUT_F), jnp.float32),
    )(aggt, W, b)


_mesh = plsc.VectorSubcoreMesh(
    core_axis_name="c", subcore_axis_name="s", num_cores=NC, num_subcores=NS
)


@functools.partial(
    pl.kernel,
    out_type=jax.ShapeDtypeStruct((IN_F * N_NODES,), jnp.float32),
    mesh=_mesh,
    compiler_params=pltpu.CompilerParams(needs_layout_passes=False),
    scratch_types=[
        pltpu.VMEM((TBL,), jnp.int32),
        pltpu.VMEM((ACC,), jnp.float32),
        pltpu.VMEM((CHUNK,), jnp.int32),
        pltpu.VMEM((CHUNK,), jnp.float32),
        pltpu.VMEM((CHUNK,), jnp.int32),
        pltpu.VMEM((CHUNK,), jnp.float32),
        pltpu.SemaphoreType.DMA,
        pltpu.SemaphoreType.DMA,
        pltpu.SemaphoreType.DMA,
        pltpu.SemaphoreType.DMA,
    ],
)
def _sc_agg(sup_hbm, pidx_hbm, ew_hbm, out_hbm,
            table_v, acc_v,
            pidx0, ew0, pidx1, ew1,
            sem_p0, sem_w0, sem_p1, sem_w1):
    cid = lax.axis_index("c")
    sid = lax.axis_index("s")
    wid = sid * NC + cid

    pltpu.sync_copy(sup_hbm.at[pl.ds(wid * TBL, TBL)], table_v)

    zvec = jnp.zeros((L,), jnp.float32)

    @pl.loop(0, ACC // L)
    def _init(i):
        acc_v[pl.ds(i * L, L)] = zvec

    slots = (
        (pidx0, ew0, sem_p0, sem_w0),
        (pidx1, ew1, sem_p1, sem_w1),
    )

    def start(c, slot):
        p_b, w_b, p_s, w_s = slot
        off = c * CHUNK
        pltpu.make_async_copy(pidx_hbm.at[pl.ds(off, CHUNK)], p_b, p_s).start()
        pltpu.make_async_copy(ew_hbm.at[pl.ds(off, CHUNK)], w_b, w_s).start()

    def wait(slot):
        p_b, w_b, p_s, w_s = slot
        pltpu.make_async_copy(pidx_hbm.at[pl.ds(0, CHUNK)], p_b, p_s).wait()
        pltpu.make_async_copy(ew_hbm.at[pl.ds(0, CHUNK)], w_b, w_s).wait()

    def process(slot):
        p_b, w_b = slot[:2]

        @plsc.parallel_loop(0, GROUPS, unroll=2)
        def _grp(g):
            o = g * L
            p = p_b[pl.ds(o, L)]
            w = w_b[pl.ds(o, L)]
            s = p & 0xFFFF
            d = lax.shift_right_logical(p, 16)
            for fp in range(PPT):
                si = s if fp == 0 else s + fp * N_NODES
                vp = plsc.load_gather(table_v, [si])
                vlo = plsc.bitcast(lax.shift_left(vp, 16), jnp.float32)
                vhi = plsc.bitcast(vp & jnp.int32(-65536), jnp.float32)
                dlo = d if fp == 0 else d + fp * N_NODES
                plsc.addupdate_scatter(acc_v, [dlo], vlo * w)
                plsc.addupdate_scatter(
                    acc_v, [d + (2 + fp) * N_NODES], vhi * w
                )

    start(0, slots[0])
    start(1, slots[1])

    @pl.loop(0, NCHUNK, step=2)
    def _chunk(c):
        wait(slots[0])
        process(slots[0])

        @pl.when(c + 2 < NCHUNK)
        def _():
            start(c + 2, slots[0])

        wait(slots[1])
        process(slots[1])

        @pl.when(c + 3 < NCHUNK)
        def _():
            start(c + 3, slots[1])

    pltpu.sync_copy(
        acc_v.at[pl.ds(0, 2 * N_NODES)],
        out_hbm.at[pl.ds(2 * wid * N_NODES, 2 * N_NODES)],
    )
    pltpu.sync_copy(
        acc_v.at[pl.ds(2 * N_NODES, 2 * N_NODES)],
        out_hbm.at[pl.ds((HALF + 2 * wid) * N_NODES, 2 * N_NODES)],
    )


def kernel(x, edge_index, edge_weight, W, b):
    src = edge_index[0].astype(jnp.int32)
    dst = edge_index[1].astype(jnp.int32)
    packed = src | (dst << 16)
    x_packed = _x_packed(x)
    aggt_flat = _sc_agg(
        x_packed.reshape(-1), packed, edge_weight.astype(jnp.float32)
    )
    return _out_matmul(aggt_flat.reshape(IN_F, N_NODES), W, b)


# speedup vs baseline: 1.0482x; 1.0069x over previous
import functools

import jax
import jax.numpy as jnp
from jax import lax
from jax.experimental import pallas as pl
from jax.experimental.pallas import tpu as pltpu
from jax.experimental.pallas import tpu_sc as plsc

N_NODES = 10000
IN_F = 128
OUT_F = 128
N_EDGES = 320000

NC = 2
NS = 16
L = 16
NW = NC * NS
FPT = IN_F // NW
PPT = FPT // 2
HALF = IN_F // 2
CHUNK = 3200
NCHUNK = N_EDGES // CHUNK
GROUPS = CHUNK // L
TBL = PPT * N_NODES
ACC = FPT * N_NODES


def _pack_body(i_ref, x_ref, o_ref):
    xt = lax.dot_general(
        i_ref[...],
        x_ref[...],
        dimension_numbers=(((1,), (1,)), ((), ())),
        preferred_element_type=jnp.float32,
    )
    lo = lax.bitcast_convert_type(
        xt[:HALF].astype(jnp.bfloat16), jnp.uint16
    ).astype(jnp.uint32)
    hi = lax.bitcast_convert_type(
        xt[HALF:].astype(jnp.bfloat16), jnp.uint16
    ).astype(jnp.uint32)
    o_ref[...] = lax.bitcast_convert_type(lo | (hi << 16), jnp.int32)


def _x_packed(x):
    n = x.shape[0]
    eye = jnp.eye(IN_F, dtype=jnp.float32)
    return pl.pallas_call(
        _pack_body,
        out_shape=jax.ShapeDtypeStruct((HALF, n), jnp.int32),
    )(eye, x)


def _out_body(aggt_ref, w_ref, b_ref, o_ref):
    o_ref[...] = lax.dot_general(
        aggt_ref[...],
        w_ref[...],
        dimension_numbers=(((0,), (0,)), ((), ())),
        preferred_element_type=jnp.float32,
    ) + b_ref[...][None, :]


def _out_matmul(aggt, W, b):
    return pl.pallas_call(
        _out_body,
        out_shape=jax.ShapeDtypeStruct((N_NODES, OUT_F), jnp.float32),
    )(aggt, W, b)


_mesh = plsc.VectorSubcoreMesh(
    core_axis_name="c", subcore_axis_name="s", num_cores=NC, num_subcores=NS
)


@functools.partial(
    pl.kernel,
    out_type=jax.ShapeDtypeStruct((IN_F * N_NODES,), jnp.float32),
    mesh=_mesh,
    compiler_params=pltpu.CompilerParams(needs_layout_passes=False),
    scratch_types=[
        pltpu.VMEM((TBL,), jnp.int32),
        pltpu.VMEM((ACC,), jnp.float32),
        pltpu.VMEM((CHUNK,), jnp.int32),
        pltpu.VMEM((CHUNK,), jnp.float32),
        pltpu.VMEM((CHUNK,), jnp.int32),
        pltpu.VMEM((CHUNK,), jnp.float32),
        pltpu.SemaphoreType.DMA,
        pltpu.SemaphoreType.DMA,
        pltpu.SemaphoreType.DMA,
        pltpu.SemaphoreType.DMA,
    ],
)
def _sc_agg(sup_hbm, pidx_hbm, ew_hbm, out_hbm,
            table_v, acc_v,
            pidx0, ew0, pidx1, ew1,
            sem_p0, sem_w0, sem_p1, sem_w1):
    cid = lax.axis_index("c")
    sid = lax.axis_index("s")
    wid = sid * NC + cid

    pltpu.sync_copy(sup_hbm.at[pl.ds(wid * TBL, TBL)], table_v)

    zvec = jnp.zeros((L,), jnp.float32)

    @pl.loop(0, ACC // L)
    def _init(i):
        acc_v[pl.ds(i * L, L)] = zvec

    slots = (
        (pidx0, ew0, sem_p0, sem_w0),
        (pidx1, ew1, sem_p1, sem_w1),
    )

    def start(c, slot):
        p_b, w_b, p_s, w_s = slot
        off = c * CHUNK
        pltpu.make_async_copy(pidx_hbm.at[pl.ds(off, CHUNK)], p_b, p_s).start()
        pltpu.make_async_copy(ew_hbm.at[pl.ds(off, CHUNK)], w_b, w_s).start()

    def wait(slot):
        p_b, w_b, p_s, w_s = slot
        pltpu.make_async_copy(pidx_hbm.at[pl.ds(0, CHUNK)], p_b, p_s).wait()
        pltpu.make_async_copy(ew_hbm.at[pl.ds(0, CHUNK)], w_b, w_s).wait()

    def process(slot):
        p_b, w_b = slot[:2]

        @plsc.parallel_loop(0, GROUPS, unroll=1)
        def _grp(g):
            o = g * L
            p = p_b[pl.ds(o, L)]
            w = w_b[pl.ds(o, L)]
            s = p & 0xFFFF
            d = lax.shift_right_logical(p, 16)
            for fp in range(PPT):
                si = s if fp == 0 else s + fp * N_NODES
                vp = plsc.load_gather(table_v, [si])
                vlo = plsc.bitcast(lax.shift_left(vp, 16), jnp.float32)
                vhi = plsc.bitcast(vp & jnp.int32(-65536), jnp.float32)
                dlo = d if fp == 0 else d + fp * N_NODES
                plsc.addupdate_scatter(acc_v, [dlo], vlo * w)
                plsc.addupdate_scatter(
                    acc_v, [d + (2 + fp) * N_NODES], vhi * w
                )

    start(0, slots[0])
    start(1, slots[1])

    @pl.loop(0, NCHUNK, step=2)
    def _chunk(c):
        wait(slots[0])
        process(slots[0])

        @pl.when(c + 2 < NCHUNK)
        def _():
            start(c + 2, slots[0])

        wait(slots[1])
        process(slots[1])

        @pl.when(c + 3 < NCHUNK)
        def _():
            start(c + 3, slots[1])

    pltpu.sync_copy(
        acc_v.at[pl.ds(0, 2 * N_NODES)],
        out_hbm.at[pl.ds(2 * wid * N_NODES, 2 * N_NODES)],
    )
    pltpu.sync_copy(
        acc_v.at[pl.ds(2 * N_NODES, 2 * N_NODES)],
        out_hbm.at[pl.ds((HALF + 2 * wid) * N_NODES, 2 * N_NODES)],
    )


def kernel(x, edge_index, edge_weight, W, b):
    src = edge_index[0].astype(jnp.int32)
    dst = edge_index[1].astype(jnp.int32)
    packed = src | (dst << 16)
    x_packed = _x_packed(x)
    aggt_flat = _sc_agg(
        x_packed.reshape(-1), packed, edge_weight.astype(jnp.float32)
    )
    return _out_matmul(aggt_flat.reshape(IN_F, N_NODES), W, b)


# speedup vs baseline: 1.0962x; 1.0458x over previous
import functools

import jax
import jax.numpy as jnp
from jax import lax
from jax.experimental import pallas as pl
from jax.experimental.pallas import tpu as pltpu
from jax.experimental.pallas import tpu_sc as plsc

N_NODES = 10000
IN_F = 128
OUT_F = 128
N_EDGES = 320000

NC = 2
NS = 16
L = 16
NW = NC * NS
FPT = IN_F // NW
PPT = FPT // 2
HALF = IN_F // 2
CHUNK = 3200
NCHUNK = N_EDGES // CHUNK
GROUPS = CHUNK // L
TBL = PPT * N_NODES
ACC = FPT * N_NODES


_EROWS = N_EDGES // 128


def _pack_body(i_ref, x_ref, ei_ref, o_ref, pidx_ref):
    xt = lax.dot_general(
        i_ref[...],
        x_ref[...],
        dimension_numbers=(((1,), (1,)), ((), ())),
        preferred_element_type=jnp.float32,
    )
    lo = lax.bitcast_convert_type(
        xt[:HALF].astype(jnp.bfloat16), jnp.uint16
    ).astype(jnp.uint32)
    hi = lax.bitcast_convert_type(
        xt[HALF:].astype(jnp.bfloat16), jnp.uint16
    ).astype(jnp.uint32)
    o_ref[...] = lax.bitcast_convert_type(lo | (hi << 16), jnp.int32)
    ei = ei_ref[...]
    pidx_ref[...] = ei[:_EROWS] | (ei[_EROWS:] << 16)


def _x_packed(x, ei2d):
    n = x.shape[0]
    eye = jnp.eye(IN_F, dtype=jnp.float32)
    return pl.pallas_call(
        _pack_body,
        out_shape=(
            jax.ShapeDtypeStruct((HALF, n), jnp.int32),
            jax.ShapeDtypeStruct((_EROWS, 128), jnp.int32),
        ),
    )(eye, x, ei2d)


def _out_body(aggt_ref, w_ref, b_ref, o_ref):
    o_ref[...] = lax.dot_general(
        aggt_ref[...],
        w_ref[...],
        dimension_numbers=(((0,), (0,)), ((), ())),
        preferred_element_type=jnp.float32,
    ) + b_ref[...][None, :]


def _out_matmul(aggt, W, b):
    return pl.pallas_call(
        _out_body,
        out_shape=jax.ShapeDtypeStruct((N_NODES, OUT_F), jnp.float32),
    )(aggt, W, b)


_mesh = plsc.VectorSubcoreMesh(
    core_axis_name="c", subcore_axis_name="s", num_cores=NC, num_subcores=NS
)


@functools.partial(
    pl.kernel,
    out_type=jax.ShapeDtypeStruct((IN_F * N_NODES,), jnp.float32),
    mesh=_mesh,
    compiler_params=pltpu.CompilerParams(needs_layout_passes=False),
    scratch_types=[
        pltpu.VMEM((TBL,), jnp.int32),
        pltpu.VMEM((ACC,), jnp.float32),
        pltpu.VMEM((CHUNK,), jnp.int32),
        pltpu.VMEM((CHUNK,), jnp.float32),
        pltpu.VMEM((CHUNK,), jnp.int32),
        pltpu.VMEM((CHUNK,), jnp.float32),
        pltpu.SemaphoreType.DMA,
        pltpu.SemaphoreType.DMA,
        pltpu.SemaphoreType.DMA,
        pltpu.SemaphoreType.DMA,
    ],
)
def _sc_agg(sup_hbm, pidx_hbm, ew_hbm, out_hbm,
            table_v, acc_v,
            pidx0, ew0, pidx1, ew1,
            sem_p0, sem_w0, sem_p1, sem_w1):
    cid = lax.axis_index("c")
    sid = lax.axis_index("s")
    wid = sid * NC + cid

    pltpu.sync_copy(sup_hbm.at[pl.ds(wid * TBL, TBL)], table_v)

    zvec = jnp.zeros((L,), jnp.float32)

    @pl.loop(0, ACC // L)
    def _init(i):
        acc_v[pl.ds(i * L, L)] = zvec

    slots = (
        (pidx0, ew0, sem_p0, sem_w0),
        (pidx1, ew1, sem_p1, sem_w1),
    )

    def start(c, slot):
        p_b, w_b, p_s, w_s = slot
        off = c * CHUNK
        pltpu.make_async_copy(pidx_hbm.at[pl.ds(off, CHUNK)], p_b, p_s).start()
        pltpu.make_async_copy(ew_hbm.at[pl.ds(off, CHUNK)], w_b, w_s).start()

    def wait(slot):
        p_b, w_b, p_s, w_s = slot
        pltpu.make_async_copy(pidx_hbm.at[pl.ds(0, CHUNK)], p_b, p_s).wait()
        pltpu.make_async_copy(ew_hbm.at[pl.ds(0, CHUNK)], w_b, w_s).wait()

    def process(slot):
        p_b, w_b = slot[:2]

        @plsc.parallel_loop(0, GROUPS, unroll=1)
        def _grp(g):
            o = g * L
            p = p_b[pl.ds(o, L)]
            w = w_b[pl.ds(o, L)]
            s = p & 0xFFFF
            d = lax.shift_right_logical(p, 16)
            for fp in range(PPT):
                si = s if fp == 0 else s + fp * N_NODES
                vp = plsc.load_gather(table_v, [si])
                vlo = plsc.bitcast(lax.shift_left(vp, 16), jnp.float32)
                vhi = plsc.bitcast(vp & jnp.int32(-65536), jnp.float32)
                dlo = d if fp == 0 else d + fp * N_NODES
                plsc.addupdate_scatter(acc_v, [dlo], vlo * w)
                plsc.addupdate_scatter(
                    acc_v, [d + (2 + fp) * N_NODES], vhi * w
                )

    start(0, slots[0])
    start(1, slots[1])

    @pl.loop(0, NCHUNK, step=2)
    def _chunk(c):
        wait(slots[0])
        process(slots[0])

        @pl.when(c + 2 < NCHUNK)
        def _():
            start(c + 2, slots[0])

        wait(slots[1])
        process(slots[1])

        @pl.when(c + 3 < NCHUNK)
        def _():
            start(c + 3, slots[1])

    pltpu.sync_copy(
        acc_v.at[pl.ds(0, 2 * N_NODES)],
        out_hbm.at[pl.ds(2 * wid * N_NODES, 2 * N_NODES)],
    )
    pltpu.sync_copy(
        acc_v.at[pl.ds(2 * N_NODES, 2 * N_NODES)],
        out_hbm.at[pl.ds((HALF + 2 * wid) * N_NODES, 2 * N_NODES)],
    )


def kernel(x, edge_index, edge_weight, W, b):
    ei2d = edge_index.astype(jnp.int32).reshape(2 * _EROWS, 128)
    x_packed, pidx = _x_packed(x, ei2d)
    aggt_flat = _sc_agg(
        x_packed.reshape(-1), pidx.reshape(-1),
        edge_weight.astype(jnp.float32),
    )
    return _out_matmul(aggt_flat.reshape(IN_F, N_NODES), W, b)


# speedup vs baseline: 1.0973x; 1.0010x over previous
import functools

import jax
import jax.numpy as jnp
from jax import lax
from jax.experimental import pallas as pl
from jax.experimental.pallas import tpu as pltpu
from jax.experimental.pallas import tpu_sc as plsc

N_NODES = 10000
IN_F = 128
OUT_F = 128
N_EDGES = 320000

NC = 2
NS = 16
L = 16
NW = NC * NS
FPT = IN_F // NW
PPT = FPT // 2
HALF = IN_F // 2
CHUNK = 6400
NCHUNK = N_EDGES // CHUNK
GROUPS = CHUNK // L
TBL = PPT * N_NODES
ACC = FPT * N_NODES


_EROWS = N_EDGES // 128


def _pack_body(i_ref, x_ref, ei_ref, o_ref, pidx_ref):
    xt = lax.dot_general(
        i_ref[...],
        x_ref[...],
        dimension_numbers=(((1,), (1,)), ((), ())),
        preferred_element_type=jnp.float32,
    )
    lo = lax.bitcast_convert_type(
        xt[:HALF].astype(jnp.bfloat16), jnp.uint16
    ).astype(jnp.uint32)
    hi = lax.bitcast_convert_type(
        xt[HALF:].astype(jnp.bfloat16), jnp.uint16
    ).astype(jnp.uint32)
    o_ref[...] = lax.bitcast_convert_type(lo | (hi << 16), jnp.int32)
    ei = ei_ref[...]
    pidx_ref[...] = ei[:_EROWS] | (ei[_EROWS:] << 16)


def _x_packed(x, ei2d):
    n = x.shape[0]
    eye = jnp.eye(IN_F, dtype=jnp.float32)
    return pl.pallas_call(
        _pack_body,
        out_shape=(
            jax.ShapeDtypeStruct((HALF, n), jnp.int32),
            jax.ShapeDtypeStruct((_EROWS, 128), jnp.int32),
        ),
    )(eye, x, ei2d)


def _out_body(aggt_ref, w_ref, b_ref, o_ref):
    o_ref[...] = lax.dot_general(
        aggt_ref[...],
        w_ref[...],
        dimension_numbers=(((0,), (0,)), ((), ())),
        preferred_element_type=jnp.float32,
    ) + b_ref[...][None, :]


def _out_matmul(aggt, W, b):
    return pl.pallas_call(
        _out_body,
        out_shape=jax.ShapeDtypeStruct((N_NODES, OUT_F), jnp.float32),
    )(aggt, W, b)


_mesh = plsc.VectorSubcoreMesh(
    core_axis_name="c", subcore_axis_name="s", num_cores=NC, num_subcores=NS
)


@functools.partial(
    pl.kernel,
    out_type=jax.ShapeDtypeStruct((IN_F * N_NODES,), jnp.float32),
    mesh=_mesh,
    compiler_params=pltpu.CompilerParams(needs_layout_passes=False),
    scratch_types=[
        pltpu.VMEM((TBL,), jnp.int32),
        pltpu.VMEM((ACC,), jnp.float32),
        pltpu.VMEM((CHUNK,), jnp.int32),
        pltpu.VMEM((CHUNK,), jnp.float32),
        pltpu.VMEM((CHUNK,), jnp.int32),
        pltpu.VMEM((CHUNK,), jnp.float32),
        pltpu.SemaphoreType.DMA,
        pltpu.SemaphoreType.DMA,
        pltpu.SemaphoreType.DMA,
        pltpu.SemaphoreType.DMA,
    ],
)
def _sc_agg(sup_hbm, pidx_hbm, ew_hbm, out_hbm,
            table_v, acc_v,
            pidx0, ew0, pidx1, ew1,
            sem_p0, sem_w0, sem_p1, sem_w1):
    cid = lax.axis_index("c")
    sid = lax.axis_index("s")
    wid = sid * NC + cid

    pltpu.sync_copy(sup_hbm.at[pl.ds(wid * TBL, TBL)], table_v)

    zvec = jnp.zeros((L,), jnp.float32)

    @pl.loop(0, ACC // L)
    def _init(i):
        acc_v[pl.ds(i * L, L)] = zvec

    slots = (
        (pidx0, ew0, sem_p0, sem_w0),
        (pidx1, ew1, sem_p1, sem_w1),
    )

    def start(c, slot):
        p_b, w_b, p_s, w_s = slot
        off = c * CHUNK
        pltpu.make_async_copy(pidx_hbm.at[pl.ds(off, CHUNK)], p_b, p_s).start()
        pltpu.make_async_copy(ew_hbm.at[pl.ds(off, CHUNK)], w_b, w_s).start()

    def wait(slot):
        p_b, w_b, p_s, w_s = slot
        pltpu.make_async_copy(pidx_hbm.at[pl.ds(0, CHUNK)], p_b, p_s).wait()
        pltpu.make_async_copy(ew_hbm.at[pl.ds(0, CHUNK)], w_b, w_s).wait()

    def process(slot):
        p_b, w_b = slot[:2]

        @plsc.parallel_loop(0, GROUPS, unroll=1)
        def _grp(g):
            o = g * L
            p = p_b[pl.ds(o, L)]
            w = w_b[pl.ds(o, L)]
            s = p & 0xFFFF
            d = lax.shift_right_logical(p, 16)
            for fp in range(PPT):
                si = s if fp == 0 else s + fp * N_NODES
                vp = plsc.load_gather(table_v, [si])
                vlo = plsc.bitcast(lax.shift_left(vp, 16), jnp.float32)
                vhi = plsc.bitcast(vp & jnp.int32(-65536), jnp.float32)
                dlo = d if fp == 0 else d + fp * N_NODES
                plsc.addupdate_scatter(acc_v, [dlo], vlo * w)
                plsc.addupdate_scatter(
                    acc_v, [d + (2 + fp) * N_NODES], vhi * w
                )

    start(0, slots[0])
    start(1, slots[1])

    @pl.loop(0, NCHUNK, step=2)
    def _chunk(c):
        wait(slots[0])
        process(slots[0])

        @pl.when(c + 2 < NCHUNK)
        def _():
            start(c + 2, slots[0])

        wait(slots[1])
        process(slots[1])

        @pl.when(c + 3 < NCHUNK)
        def _():
            start(c + 3, slots[1])

    pltpu.sync_copy(
        acc_v.at[pl.ds(0, 2 * N_NODES)],
        out_hbm.at[pl.ds(2 * wid * N_NODES, 2 * N_NODES)],
    )
    pltpu.sync_copy(
        acc_v.at[pl.ds(2 * N_NODES, 2 * N_NODES)],
        out_hbm.at[pl.ds((HALF + 2 * wid) * N_NODES, 2 * N_NODES)],
    )


def kernel(x, edge_index, edge_weight, W, b):
    ei2d = edge_index.astype(jnp.int32).reshape(2 * _EROWS, 128)
    x_packed, pidx = _x_packed(x, ei2d)
    aggt_flat = _sc_agg(
        x_packed.reshape(-1), pidx.reshape(-1),
        edge_weight.astype(jnp.float32),
    )
    return _out_matmul(aggt_flat.reshape(IN_F, N_NODES), W, b)
